# baseline (speedup 1.0000x reference)
import jax
import jax.numpy as jnp
from jax.experimental import pallas as pl

HID = 8


def _mlp3(x, p, pre, act):
    h = act(x @ p[pre + '_w0'] + p[pre + '_b0'])
    h = act(h @ p[pre + '_w1'] + p[pre + '_b1'])
    return h @ p[pre + '_w2'] + p[pre + '_b2']


def _res_layer(resources, operations, req, p, pre):
    r = resources @ p[pre + 'Wr']
    o = operations @ p[pre + 'Wo']
    src, dst = req[0], req[1]
    ops_e = o[src]
    res_e = r[dst]
    sa = jax.nn.leaky_relu(jnp.concatenate([r, r], -1) @ p[pre + 'a_self'], 0.2)
    ca = jax.nn.leaky_relu(jnp.concatenate([res_e, ops_e], -1) @ p[pre + 'a_cross'], 0.2)
    norm = jax.nn.softmax(jnp.concatenate([sa, ca], 0), axis=0)
    ns = norm[:r.shape[0]]
    nc = norm[r.shape[0]:]
    summed = jnp.zeros_like(r).at[dst].add(nc * ops_e)
    return jax.nn.elu(ns * r + summed)


def _op_layer(operations, resources, prec, req, w, p, pre):
    N = operations.shape[0]
    agg = jnp.zeros((N, resources.shape[1]), operations.dtype).at[req[0]].add(resources[req[1]])
    s, d = prec[0], prec[1]
    in_sum = jnp.zeros_like(operations).at[d].add(operations[s] * w[:, None])
    in_cnt = jnp.zeros((N,), operations.dtype).at[d].add(w)
    pred_mean = in_sum / in_cnt[:, None]
    out_sum = jnp.zeros_like(operations).at[s].add(operations[d] * w[:, None])
    out_cnt = jnp.zeros((N,), operations.dtype).at[s].add(w)
    succ_mean = out_sum / out_cnt[:, None]
    elu = jax.nn.elu
    preds = _mlp3(pred_mean[1:-1], p, pre + 'pred', elu)
    succs = _mlp3(succ_mean[1:-1], p, pre + 'succ', elu)
    same = _mlp3(operations[1:-1], p, pre + 'same', elu)
    aggm = _mlp3(agg[1:-1], p, pre + 'res', elu)
    comb = _mlp3(jnp.concatenate([preds, succs, aggm, same], -1), p, pre + 'comb', elu)
    return jnp.zeros((N, HID), operations.dtype).at[1:-1].set(comb)


def kernel(operations, resources, precedence_edges, requirement_edges, actions, params):
    p = params
    prec, req = precedence_edges, requirement_edges
    N = operations.shape[0]
    E = prec.shape[1]
    keys = prec[0] * N + prec[1]
    eids = jnp.arange(E, dtype=jnp.int32)
    table = jnp.zeros((N * N,), jnp.int32).at[keys].max(eids)
    w = (table[keys] == eids).astype(jnp.float32)

    ops, res = operations, resources
    for l in range(2):
        res = _res_layer(res, ops, req, p, 'r%d_' % l)
        ops = _op_layer(ops, res, prec, req, w, p, 'o%d_' % l)
    gs = jnp.concatenate([ops.mean(0), res.mean(0)])
    feat = jnp.concatenate([ops[actions[:, 0]], res[actions[:, 1]],
                            jnp.broadcast_to(gs, (actions.shape[0], gs.shape[0]))], -1)
    logits = _mlp3(feat, p, 'actor', jnp.tanh)
    value = _mlp3(gs[None, :], p, 'critic', jnp.tanh)
    return jnp.concatenate([logits[:, 0], value[:, 0]])


# speedup vs baseline: 2.2148x; 2.2148x over previous
import functools

import jax
import jax.numpy as jnp
from jax import lax
from jax.experimental import pallas as pl
from jax.experimental.pallas import tpu as pltpu
from jax.experimental.pallas import tpu_sc as plsc

HID = 8
_NC, _NS = 2, 16


def _sc_agg(res_flat, req_src, req_dst):
    E = req_src.shape[0]
    EPT = E // (_NC * _NS)
    CH = 400
    NROW = CH * 8 // 128
    ACCW = 16384
    mesh = plsc.VectorSubcoreMesh(core_axis_name="c", subcore_axis_name="s")

    @functools.partial(
        pl.kernel, mesh=mesh,
        compiler_params=pltpu.CompilerParams(needs_layout_passes=False),
        out_type=jax.ShapeDtypeStruct((_NC * ACCW,), jnp.float32),
        scratch_types=dict(
            tab=pltpu.VMEM((2000 * 8,), jnp.float32),
            sbuf=pltpu.VMEM((CH,), jnp.int32),
            dbuf=pltpu.VMEM((CH,), jnp.int32),
            idxb=pltpu.VMEM((NROW * 128,), jnp.int32),
            valb=pltpu.VMEM((NROW * 128,), jnp.float32),
            zbuf=pltpu.VMEM((ACCW // _NS,), jnp.float32),
            acc=pltpu.VMEM_SHARED((ACCW,), jnp.float32),
        ),
    )
    def k(res_hbm, src_hbm, dst_hbm, out_hbm, tab, sbuf, dbuf, idxb, valb, zbuf, acc):
        c = lax.axis_index("c")
        s = lax.axis_index("s")
        base = (c * _NS + s) * EPT
        pltpu.sync_copy(res_hbm, tab)
        zero16 = jnp.zeros((16,), jnp.float32)
        zslice = ACCW // _NS

        @pl.loop(0, zslice // 16)
        def _(i):
            zbuf[pl.ds(i * 16, 16)] = zero16

        pltpu.sync_copy(zbuf, acc.at[pl.ds(s * zslice, zslice)])
        plsc.subcore_barrier()
        iota = lax.iota(jnp.int32, 16)

        @pl.loop(0, EPT // CH)
        def _(kk):
            off = base + kk * CH
            pltpu.sync_copy(src_hbm.at[pl.ds(off, CH)], sbuf)
            pltpu.sync_copy(dst_hbm.at[pl.ds(off, CH)], dbuf)

            @pl.loop(0, CH // 16)
            def _(g):
                sv = sbuf[pl.ds(g * 16, 16)]
                dv = dbuf[pl.ds(g * 16, 16)]
                for j in range(8):
                    vals = plsc.load_gather(tab, [dv * 8 + j])
                    pos = g * 128 + iota * 8 + j
                    plsc.store_scatter(valb, [pos], vals)
                    plsc.store_scatter(idxb, [pos], sv * 8 + j)

            pltpu.sync_copy(valb, acc.at[idxb], add=True)

        plsc.subcore_barrier()
        pltpu.sync_copy(acc.at[pl.ds(s * zslice, zslice)],
                        out_hbm.at[pl.ds(c * ACCW + s * zslice, zslice)])

    return k(res_flat, req_src, req_dst)


def _sc_rep_scatter(prec_src, prec_dst, n):
    E = prec_src.shape[0]
    EPT = E // (_NC * _NS)
    CH = 400
    mesh = plsc.VectorSubcoreMesh(core_axis_name="c", subcore_axis_name="s")

    @functools.partial(
        pl.kernel, mesh=mesh,
        compiler_params=pltpu.CompilerParams(needs_layout_passes=False),
        out_type=jax.ShapeDtypeStruct((n * n,), jnp.int32),
        scratch_types=dict(
            sbuf=pltpu.VMEM((CH,), jnp.int32),
            dbuf=pltpu.VMEM((CH,), jnp.int32),
            kbuf=pltpu.VMEM((CH,), jnp.int32),
            ebuf=pltpu.VMEM((CH,), jnp.int32),
        ),
    )
    def k(src_hbm, dst_hbm, out_hbm, sbuf, dbuf, kbuf, ebuf):
        c = lax.axis_index("c")
        s = lax.axis_index("s")
        base = (c * _NS + s) * EPT
        iota = lax.iota(jnp.int32, 16)

        @pl.loop(0, EPT // CH)
        def _(kk):
            off = base + kk * CH
            pltpu.sync_copy(src_hbm.at[pl.ds(off, CH)], sbuf)
            pltpu.sync_copy(dst_hbm.at[pl.ds(off, CH)], dbuf)

            @pl.loop(0, CH // 16)
            def _(g):
                sv = sbuf[pl.ds(g * 16, 16)]
                dv = dbuf[pl.ds(g * 16, 16)]
                kbuf[pl.ds(g * 16, 16)] = sv * n + dv
                ebuf[pl.ds(g * 16, 16)] = off + g * 16 + iota

            pltpu.sync_copy(ebuf, out_hbm.at[kbuf])

    return k(prec_src, prec_dst)


def _sc_prec(ops_flat, prec_src, prec_dst, rep_tab, n, f):
    E = prec_src.shape[0]
    EPT = E // (_NC * _NS)
    CH = 400
    NP = 10240
    NP8 = 81920
    SEG = 2 * NP8 + 2 * NP
    mesh = plsc.VectorSubcoreMesh(core_axis_name="c", subcore_axis_name="s")

    @functools.partial(
        pl.kernel, mesh=mesh,
        compiler_params=pltpu.CompilerParams(needs_layout_passes=False),
        out_type=jax.ShapeDtypeStruct((_NC * SEG,), jnp.float32),
        scratch_types=dict(
            tab=pltpu.VMEM((n * f,), jnp.float32),
            sbuf=pltpu.VMEM((CH,), jnp.int32),
            dbuf=pltpu.VMEM((CH,), jnp.int32),
            kbuf=pltpu.VMEM((CH,), jnp.int32),
            rbuf=pltpu.VMEM((CH,), jnp.int32),
            iv=pltpu.VMEM((CH * f,), jnp.int32),
            vv=pltpu.VMEM((CH * f,), jnp.float32),
            ov=pltpu.VMEM((CH * f,), jnp.int32),
            wv=pltpu.VMEM((CH * f,), jnp.float32),
            ic=pltpu.VMEM((2 * CH,), jnp.int32),
            vc=pltpu.VMEM((2 * CH,), jnp.float32),
            zbuf=pltpu.VMEM((SEG // _NS,), jnp.float32),
            acc=pltpu.VMEM_SHARED((SEG,), jnp.float32),
        ),
    )
    def k(ops_hbm, src_hbm, dst_hbm, rep_hbm, out_hbm,
          tab, sbuf, dbuf, kbuf, rbuf, iv, vv, ov, wv, ic, vc, zbuf, acc):
        c = lax.axis_index("c")
        s = lax.axis_index("s")
        base = (c * _NS + s) * EPT
        pltpu.sync_copy(ops_hbm, tab)
        zero16 = jnp.zeros((16,), jnp.float32)
        zslice = SEG // _NS

        @pl.loop(0, zslice // 16)
        def _(i):
            zbuf[pl.ds(i * 16, 16)] = zero16

        pltpu.sync_copy(zbuf, acc.at[pl.ds(s * zslice, zslice)])
        plsc.subcore_barrier()
        iota = lax.iota(jnp.int32, 16)

        @pl.loop(0, EPT // CH)
        def _(kk):
            off = base + kk * CH
            pltpu.sync_copy(src_hbm.at[pl.ds(off, CH)], sbuf)
            pltpu.sync_copy(dst_hbm.at[pl.ds(off, CH)], dbuf)

            @pl.loop(0, CH // 16)
            def _(g):
                sv = sbuf[pl.ds(g * 16, 16)]
                dv = dbuf[pl.ds(g * 16, 16)]
                kbuf[pl.ds(g * 16, 16)] = sv * n + dv

            pltpu.sync_copy(rep_hbm.at[kbuf], rbuf)

            @pl.loop(0, CH // 16)
            def _(g):
                sv = sbuf[pl.ds(g * 16, 16)]
                dv = dbuf[pl.ds(g * 16, 16)]
                rv = rbuf[pl.ds(g * 16, 16)]
                eid = off + g * 16 + iota
                w = jnp.where(rv == eid, 1.0, 0.0).astype(jnp.float32)
                for j in range(f):
                    pos = g * 16 * f + iota * f + j
                    vs = plsc.load_gather(tab, [sv * f + j]) * w
                    plsc.store_scatter(wv, [pos], vs)
                    plsc.store_scatter(iv, [pos], dv * f + j)
                    vd = plsc.load_gather(tab, [dv * f + j]) * w
                    plsc.store_scatter(vv, [pos], vd)
                    plsc.store_scatter(ov, [pos], NP8 + sv * f + j)
                cpos = g * 32 + iota * 2
                plsc.store_scatter(vc, [cpos], w)
                plsc.store_scatter(ic, [cpos], 2 * NP8 + dv)
                plsc.store_scatter(vc, [cpos + 1], w)
                plsc.store_scatter(ic, [cpos + 1], 2 * NP8 + NP + sv)

            pltpu.sync_copy(wv, acc.at[iv], add=True)
            pltpu.sync_copy(vv, acc.at[ov], add=True)
            pltpu.sync_copy(vc, acc.at[ic], add=True)

        plsc.subcore_barrier()
        pltpu.sync_copy(acc.at[pl.ds(s * zslice, zslice)],
                        out_hbm.at[pl.ds(c * SEG + s * zslice, zslice)])

    return k(ops_flat, prec_src, prec_dst, rep_tab)


def _mlp3(x, p, pre, act):
    h = act(x @ p[pre + '_w0'] + p[pre + '_b0'])
    h = act(h @ p[pre + '_w1'] + p[pre + '_b1'])
    return h @ p[pre + '_w2'] + p[pre + '_b2']


def _res_layer(resources, operations, req, p, pre):
    r = resources @ p[pre + 'Wr']
    o = operations @ p[pre + 'Wo']
    src, dst = req[0], req[1]
    ops_e = o[src]
    res_e = r[dst]
    sa = jax.nn.leaky_relu(jnp.concatenate([r, r], -1) @ p[pre + 'a_self'], 0.2)
    ca = jax.nn.leaky_relu(jnp.concatenate([res_e, ops_e], -1) @ p[pre + 'a_cross'], 0.2)
    norm = jax.nn.softmax(jnp.concatenate([sa, ca], 0), axis=0)
    ns = norm[:r.shape[0]]
    nc = norm[r.shape[0]:]
    summed = jnp.zeros_like(r).at[dst].add(nc * ops_e)
    return jax.nn.elu(ns * r + summed)


def _op_layer(operations, resources, prec, req, rep, p, pre):
    N, F = operations.shape
    parts = _sc_agg(resources.reshape(-1), req[0], req[1])
    agg2000 = (parts[:16000] + parts[16384:16384 + 16000]).reshape(2000, 8)
    agg = jnp.concatenate(
        [agg2000, jnp.zeros((N - 2000, 8), operations.dtype)], axis=0)
    NP, NP8 = 10240, 81920
    SEG = 2 * NP8 + 2 * NP
    pp = _sc_prec(operations.reshape(-1), prec[0], prec[1], rep, N, F)
    tot = pp[:SEG] + pp[SEG:]
    in_sum = tot[0:N * F].reshape(N, F)
    out_sum = tot[NP8:NP8 + N * F].reshape(N, F)
    in_cnt = tot[2 * NP8:2 * NP8 + N]
    out_cnt = tot[2 * NP8 + NP:2 * NP8 + NP + N]
    pred_mean = in_sum / in_cnt[:, None]
    succ_mean = out_sum / out_cnt[:, None]
    elu = jax.nn.elu
    preds = _mlp3(pred_mean[1:-1], p, pre + 'pred', elu)
    succs = _mlp3(succ_mean[1:-1], p, pre + 'succ', elu)
    same = _mlp3(operations[1:-1], p, pre + 'same', elu)
    aggm = _mlp3(agg[1:-1], p, pre + 'res', elu)
    comb = _mlp3(jnp.concatenate([preds, succs, aggm, same], -1), p, pre + 'comb', elu)
    return jnp.zeros((N, HID), operations.dtype).at[1:-1].set(comb)


def kernel(operations, resources, precedence_edges, requirement_edges, actions, params):
    p = params
    prec, req = precedence_edges, requirement_edges
    N = operations.shape[0]
    rep = _sc_rep_scatter(prec[0], prec[1], N)

    ops, res = operations, resources
    for l in range(2):
        res = _res_layer(res, ops, req, p, 'r%d_' % l)
        ops = _op_layer(ops, res, prec, req, rep, p, 'o%d_' % l)
    gs = jnp.concatenate([ops.mean(0), res.mean(0)])
    feat = jnp.concatenate([ops[actions[:, 0]], res[actions[:, 1]],
                            jnp.broadcast_to(gs, (actions.shape[0], gs.shape[0]))], -1)
    logits = _mlp3(feat, p, 'actor', jnp.tanh)
    value = _mlp3(gs[None, :], p, 'critic', jnp.tanh)
    return jnp.concatenate([logits[:, 0], value[:, 0]])


# speedup vs baseline: 12.8684x; 5.8103x over previous
import functools

import jax
import jax.numpy as jnp
from jax import lax
from jax.experimental import pallas as pl
from jax.experimental.pallas import tpu as pltpu
from jax.experimental.pallas import tpu_sc as plsc

HID = 8
_NC, _NS = 2, 16


def _sc_agg(res_flat, req_src, req_dst):
    E = req_src.shape[0]
    EPT = E // (_NC * _NS)
    CH = 400
    NROW = CH * 8 // 128
    ACCW = 16384
    mesh = plsc.VectorSubcoreMesh(core_axis_name="c", subcore_axis_name="s")

    @functools.partial(
        pl.kernel, mesh=mesh,
        compiler_params=pltpu.CompilerParams(needs_layout_passes=False),
        out_type=jax.ShapeDtypeStruct((_NC * ACCW,), jnp.float32),
        scratch_types=dict(
            tab=pltpu.VMEM((2000 * 8,), jnp.float32),
            sbuf=pltpu.VMEM((CH,), jnp.int32),
            dbuf=pltpu.VMEM((CH,), jnp.int32),
            idxb=pltpu.VMEM((NROW * 128,), jnp.int32),
            valb=pltpu.VMEM((NROW * 128,), jnp.float32),
            zbuf=pltpu.VMEM((ACCW // _NS,), jnp.float32),
            acc=pltpu.VMEM_SHARED((ACCW,), jnp.float32),
        ),
    )
    def k(res_hbm, src_hbm, dst_hbm, out_hbm, tab, sbuf, dbuf, idxb, valb, zbuf, acc):
        c = lax.axis_index("c")
        s = lax.axis_index("s")
        base = (c * _NS + s) * EPT
        pltpu.sync_copy(res_hbm, tab)
        zero16 = jnp.zeros((16,), jnp.float32)
        zslice = ACCW // _NS

        @pl.loop(0, zslice // 16)
        def _(i):
            zbuf[pl.ds(i * 16, 16)] = zero16

        pltpu.sync_copy(zbuf, acc.at[pl.ds(s * zslice, zslice)])
        plsc.subcore_barrier()
        iota = lax.iota(jnp.int32, 16)

        @pl.loop(0, EPT // CH)
        def _(kk):
            off = base + kk * CH
            pltpu.sync_copy(src_hbm.at[pl.ds(off, CH)], sbuf)
            pltpu.sync_copy(dst_hbm.at[pl.ds(off, CH)], dbuf)

            @pl.loop(0, CH // 16)
            def _(g):
                sv = sbuf[pl.ds(g * 16, 16)]
                dv = dbuf[pl.ds(g * 16, 16)]
                for j in range(8):
                    vals = plsc.load_gather(tab, [dv * 8 + j])
                    pos = g * 128 + iota * 8 + j
                    plsc.store_scatter(valb, [pos], vals)
                    plsc.store_scatter(idxb, [pos], sv * 8 + j)

            pltpu.sync_copy(valb, acc.at[idxb], add=True)

        plsc.subcore_barrier()
        pltpu.sync_copy(acc.at[pl.ds(s * zslice, zslice)],
                        out_hbm.at[pl.ds(c * ACCW + s * zslice, zslice)])

    return k(res_flat, req_src, req_dst)


def _sc_rep_scatter(prec_src, prec_dst, n):
    E = prec_src.shape[0]
    EPT = E // (_NC * _NS)
    CH = 400
    mesh = plsc.VectorSubcoreMesh(core_axis_name="c", subcore_axis_name="s")

    @functools.partial(
        pl.kernel, mesh=mesh,
        compiler_params=pltpu.CompilerParams(needs_layout_passes=False),
        out_type=jax.ShapeDtypeStruct((n * n,), jnp.int32),
        scratch_types=dict(
            sbuf=pltpu.VMEM((CH,), jnp.int32),
            dbuf=pltpu.VMEM((CH,), jnp.int32),
            kbuf=pltpu.VMEM((CH,), jnp.int32),
            ebuf=pltpu.VMEM((CH,), jnp.int32),
        ),
    )
    def k(src_hbm, dst_hbm, out_hbm, sbuf, dbuf, kbuf, ebuf):
        c = lax.axis_index("c")
        s = lax.axis_index("s")
        base = (c * _NS + s) * EPT
        iota = lax.iota(jnp.int32, 16)

        @pl.loop(0, EPT // CH)
        def _(kk):
            off = base + kk * CH
            pltpu.sync_copy(src_hbm.at[pl.ds(off, CH)], sbuf)
            pltpu.sync_copy(dst_hbm.at[pl.ds(off, CH)], dbuf)

            @pl.loop(0, CH // 16)
            def _(g):
                sv = sbuf[pl.ds(g * 16, 16)]
                dv = dbuf[pl.ds(g * 16, 16)]
                kbuf[pl.ds(g * 16, 16)] = sv * n + dv
                ebuf[pl.ds(g * 16, 16)] = off + g * 16 + iota

            pltpu.sync_copy(ebuf, out_hbm.at[kbuf])

    return k(prec_src, prec_dst)


def _sc_prec(ops_flat, prec_src, prec_dst, rep_tab, n, f):
    E = prec_src.shape[0]
    EPT = E // (_NC * _NS)
    CH = 400
    NP = 10240
    NP8 = 81920
    SEG = 2 * NP8 + 2 * NP
    mesh = plsc.VectorSubcoreMesh(core_axis_name="c", subcore_axis_name="s")

    @functools.partial(
        pl.kernel, mesh=mesh,
        compiler_params=pltpu.CompilerParams(needs_layout_passes=False),
        out_type=jax.ShapeDtypeStruct((_NC * SEG,), jnp.float32),
        scratch_types=dict(
            tab=pltpu.VMEM((n * f,), jnp.float32),
            sbuf=pltpu.VMEM((CH,), jnp.int32),
            dbuf=pltpu.VMEM((CH,), jnp.int32),
            kbuf=pltpu.VMEM((CH,), jnp.int32),
            rbuf=pltpu.VMEM((CH,), jnp.int32),
            iv=pltpu.VMEM((CH * f,), jnp.int32),
            vv=pltpu.VMEM((CH * f,), jnp.float32),
            ov=pltpu.VMEM((CH * f,), jnp.int32),
            wv=pltpu.VMEM((CH * f,), jnp.float32),
            ic=pltpu.VMEM((2 * CH,), jnp.int32),
            vc=pltpu.VMEM((2 * CH,), jnp.float32),
            zbuf=pltpu.VMEM((SEG // _NS,), jnp.float32),
            acc=pltpu.VMEM_SHARED((SEG,), jnp.float32),
        ),
    )
    def k(ops_hbm, src_hbm, dst_hbm, rep_hbm, out_hbm,
          tab, sbuf, dbuf, kbuf, rbuf, iv, vv, ov, wv, ic, vc, zbuf, acc):
        c = lax.axis_index("c")
        s = lax.axis_index("s")
        base = (c * _NS + s) * EPT
        pltpu.sync_copy(ops_hbm, tab)
        zero16 = jnp.zeros((16,), jnp.float32)
        zslice = SEG // _NS

        @pl.loop(0, zslice // 16)
        def _(i):
            zbuf[pl.ds(i * 16, 16)] = zero16

        pltpu.sync_copy(zbuf, acc.at[pl.ds(s * zslice, zslice)])
        plsc.subcore_barrier()
        iota = lax.iota(jnp.int32, 16)

        @pl.loop(0, EPT // CH)
        def _(kk):
            off = base + kk * CH
            pltpu.sync_copy(src_hbm.at[pl.ds(off, CH)], sbuf)
            pltpu.sync_copy(dst_hbm.at[pl.ds(off, CH)], dbuf)

            @pl.loop(0, CH // 16)
            def _(g):
                sv = sbuf[pl.ds(g * 16, 16)]
                dv = dbuf[pl.ds(g * 16, 16)]
                kbuf[pl.ds(g * 16, 16)] = sv * n + dv

            pltpu.sync_copy(rep_hbm.at[kbuf], rbuf)

            @pl.loop(0, CH // 16)
            def _(g):
                sv = sbuf[pl.ds(g * 16, 16)]
                dv = dbuf[pl.ds(g * 16, 16)]
                rv = rbuf[pl.ds(g * 16, 16)]
                eid = off + g * 16 + iota
                w = jnp.where(rv == eid, 1.0, 0.0).astype(jnp.float32)
                for j in range(f):
                    pos = g * 16 * f + iota * f + j
                    vs = plsc.load_gather(tab, [sv * f + j]) * w
                    plsc.store_scatter(wv, [pos], vs)
                    plsc.store_scatter(iv, [pos], dv * f + j)
                    vd = plsc.load_gather(tab, [dv * f + j]) * w
                    plsc.store_scatter(vv, [pos], vd)
                    plsc.store_scatter(ov, [pos], NP8 + sv * f + j)
                cpos = g * 32 + iota * 2
                plsc.store_scatter(vc, [cpos], w)
                plsc.store_scatter(ic, [cpos], 2 * NP8 + dv)
                plsc.store_scatter(vc, [cpos + 1], w)
                plsc.store_scatter(ic, [cpos + 1], 2 * NP8 + NP + sv)

            pltpu.sync_copy(wv, acc.at[iv], add=True)
            pltpu.sync_copy(vv, acc.at[ov], add=True)
            pltpu.sync_copy(vc, acc.at[ic], add=True)

        plsc.subcore_barrier()
        pltpu.sync_copy(acc.at[pl.ds(s * zslice, zslice)],
                        out_hbm.at[pl.ds(c * SEG + s * zslice, zslice)])

    return k(ops_flat, prec_src, prec_dst, rep_tab)


def _sc_res_edges(uc, vc, o_flat, req_src, req_dst):
    E = req_src.shape[0]
    EPT = E // (_NC * _NS)
    CH = 400
    ACCW = 16384
    SEGR = ACCW + 256 + 128
    mesh = plsc.VectorSubcoreMesh(core_axis_name="c", subcore_axis_name="s")

    @functools.partial(
        pl.kernel, mesh=mesh,
        compiler_params=pltpu.CompilerParams(needs_layout_passes=False),
        out_type=jax.ShapeDtypeStruct((_NC * SEGR,), jnp.float32),
        scratch_types=dict(
            utab=pltpu.VMEM((2000,), jnp.float32),
            vtab=pltpu.VMEM((2000,), jnp.float32),
            otab=pltpu.VMEM((16000,), jnp.float32),
            sbuf=pltpu.VMEM((CH,), jnp.int32),
            dbuf=pltpu.VMEM((CH,), jnp.int32),
            idxb=pltpu.VMEM((CH * 8,), jnp.int32),
            valb=pltpu.VMEM((CH * 8,), jnp.float32),
            mbuf=pltpu.VMEM((16 * _NS,), jnp.float32),
            zbuf=pltpu.VMEM((ACCW // _NS,), jnp.float32),
            acc=pltpu.VMEM_SHARED((ACCW,), jnp.float32),
            mtab=pltpu.VMEM_SHARED((16 * _NS,), jnp.float32),
            ztab=pltpu.VMEM_SHARED((16 * _NS,), jnp.float32),
        ),
    )
    def k(uc_hbm, vc_hbm, o_hbm, src_hbm, dst_hbm, out_hbm,
          utab, vtab, otab, sbuf, dbuf, idxb, valb, mbuf, zbuf,
          acc, mtab, ztab):
        c = lax.axis_index("c")
        s = lax.axis_index("s")
        base = (c * _NS + s) * EPT
        pltpu.sync_copy(uc_hbm, utab)
        pltpu.sync_copy(vc_hbm, vtab)
        pltpu.sync_copy(o_hbm, otab)
        zero16 = jnp.zeros((16,), jnp.float32)
        zslice = ACCW // _NS

        @pl.loop(0, zslice // 16)
        def _(i):
            zbuf[pl.ds(i * 16, 16)] = zero16

        pltpu.sync_copy(zbuf, acc.at[pl.ds(s * zslice, zslice)])
        iota = lax.iota(jnp.int32, 16)
        neg = jnp.full((16,), -3.0e38, jnp.float32)

        def ca_of(g):
            sv = sbuf[pl.ds(g * 16, 16)]
            dv = dbuf[pl.ds(g * 16, 16)]
            x = plsc.load_gather(utab, [dv]) + plsc.load_gather(vtab, [sv])
            return jnp.where(x >= 0.0, x, 0.2 * x)

        def p1_chunk(kk, mv):
            off = base + kk * CH
            pltpu.sync_copy(src_hbm.at[pl.ds(off, CH)], sbuf)
            pltpu.sync_copy(dst_hbm.at[pl.ds(off, CH)], dbuf)

            def p1_g(g, mv2):
                return jnp.maximum(mv2, ca_of(g))

            return pl.loop(0, CH // 16, init_carry=mv)(p1_g)

        mv = pl.loop(0, EPT // CH, init_carry=neg)(p1_chunk)
        mbuf[pl.ds(0, 16)] = mv
        pltpu.sync_copy(mbuf.at[pl.ds(0, 16)], mtab.at[pl.ds(s * 16, 16)])
        plsc.subcore_barrier()
        pltpu.sync_copy(mtab, mbuf)

        def mred(kidx, mv2):
            return jnp.maximum(mv2, mbuf[pl.ds(kidx * 16, 16)])

        mv = pl.loop(0, _NS, init_carry=neg)(mred)
        m = lax.reduce_max(mv, axes=(0,))

        def p2_chunk(kk, zv):
            off = base + kk * CH
            pltpu.sync_copy(src_hbm.at[pl.ds(off, CH)], sbuf)
            pltpu.sync_copy(dst_hbm.at[pl.ds(off, CH)], dbuf)

            def p2_g(g, zv2):
                sv = sbuf[pl.ds(g * 16, 16)]
                dv = dbuf[pl.ds(g * 16, 16)]
                t = jnp.exp(ca_of(g) - m)
                for j in range(8):
                    pos = g * 128 + iota * 8 + j
                    vals = plsc.load_gather(otab, [sv * 8 + j]) * t
                    plsc.store_scatter(valb, [pos], vals)
                    plsc.store_scatter(idxb, [pos], dv * 8 + j)
                return zv2 + t

            zv = pl.loop(0, CH // 16, init_carry=zv)(p2_g)
            pltpu.sync_copy(valb, acc.at[idxb], add=True)
            return zv

        zv = pl.loop(0, EPT // CH, init_carry=zero16)(p2_chunk)
        mbuf[pl.ds(0, 16)] = zv
        pltpu.sync_copy(mbuf.at[pl.ds(0, 16)], ztab.at[pl.ds(s * 16, 16)])
        plsc.subcore_barrier()
        obase = c * SEGR
        pltpu.sync_copy(acc.at[pl.ds(s * zslice, zslice)],
                        out_hbm.at[pl.ds(obase + s * zslice, zslice)])

        @pl.when(s == 0)
        def _():
            pltpu.sync_copy(ztab, out_hbm.at[pl.ds(obase + ACCW, 256)])
            mbuf[pl.ds(0, 16)] = jnp.full((16,), m, jnp.float32)
            pltpu.sync_copy(mbuf.at[pl.ds(0, 16)],
                            out_hbm.at[pl.ds(obase + ACCW + 256, 16)])

    return k(uc, vc, o_flat, req_src, req_dst)


def _mlp3(x, p, pre, act):
    h = act(x @ p[pre + '_w0'] + p[pre + '_b0'])
    h = act(h @ p[pre + '_w1'] + p[pre + '_b1'])
    return h @ p[pre + '_w2'] + p[pre + '_b2']


def _res_layer(resources, operations, req, p, pre):
    r = resources @ p[pre + 'Wr']
    o2000 = operations[:2000] @ p[pre + 'Wo']
    ac = p[pre + 'a_cross']
    uc = (r @ ac[:8]).reshape(-1)
    vc = (o2000 @ ac[8:]).reshape(-1)
    sa = jax.nn.leaky_relu(jnp.concatenate([r, r], -1) @ p[pre + 'a_self'], 0.2)
    out = _sc_res_edges(uc, vc, o2000.reshape(-1), req[0], req[1])
    SEGR = 16384 + 256 + 128
    o0, o1 = out[:SEGR], out[SEGR:]
    acc0 = o0[:16000].reshape(2000, 8)
    acc1 = o1[:16000].reshape(2000, 8)
    z0 = o0[16384:16640].sum()
    z1 = o1[16384:16640].sum()
    m0 = o0[16640]
    m1 = o1[16640]
    m = jnp.maximum(jnp.maximum(m0, m1), sa.max())
    e0 = jnp.exp(m0 - m)
    e1 = jnp.exp(m1 - m)
    esa = jnp.exp(sa - m)
    Z = z0 * e0 + z1 * e1 + esa.sum()
    summed = (acc0 * e0 + acc1 * e1) / Z
    ns = esa / Z
    return jax.nn.elu(ns * r + summed)


def _op_layer(operations, resources, prec, req, rep, p, pre):
    N, F = operations.shape
    parts = _sc_agg(resources.reshape(-1), req[0], req[1])
    agg2000 = (parts[:16000] + parts[16384:16384 + 16000]).reshape(2000, 8)
    agg = jnp.concatenate(
        [agg2000, jnp.zeros((N - 2000, 8), operations.dtype)], axis=0)
    NP, NP8 = 10240, 81920
    SEG = 2 * NP8 + 2 * NP
    pp = _sc_prec(operations.reshape(-1), prec[0], prec[1], rep, N, F)
    tot = pp[:SEG] + pp[SEG:]
    in_sum = tot[0:N * F].reshape(N, F)
    out_sum = tot[NP8:NP8 + N * F].reshape(N, F)
    in_cnt = tot[2 * NP8:2 * NP8 + N]
    out_cnt = tot[2 * NP8 + NP:2 * NP8 + NP + N]
    pred_mean = in_sum / in_cnt[:, None]
    succ_mean = out_sum / out_cnt[:, None]
    elu = jax.nn.elu
    preds = _mlp3(pred_mean[1:-1], p, pre + 'pred', elu)
    succs = _mlp3(succ_mean[1:-1], p, pre + 'succ', elu)
    same = _mlp3(operations[1:-1], p, pre + 'same', elu)
    aggm = _mlp3(agg[1:-1], p, pre + 'res', elu)
    comb = _mlp3(jnp.concatenate([preds, succs, aggm, same], -1), p, pre + 'comb', elu)
    return jnp.zeros((N, HID), operations.dtype).at[1:-1].set(comb)


def kernel(operations, resources, precedence_edges, requirement_edges, actions, params):
    p = params
    prec, req = precedence_edges, requirement_edges
    N = operations.shape[0]
    rep = _sc_rep_scatter(prec[0], prec[1], N)

    ops, res = operations, resources
    for l in range(2):
        res = _res_layer(res, ops, req, p, 'r%d_' % l)
        ops = _op_layer(ops, res, prec, req, rep, p, 'o%d_' % l)
    gs = jnp.concatenate([ops.mean(0), res.mean(0)])
    feat = jnp.concatenate([ops[actions[:, 0]], res[actions[:, 1]],
                            jnp.broadcast_to(gs, (actions.shape[0], gs.shape[0]))], -1)
    logits = _mlp3(feat, p, 'actor', jnp.tanh)
    value = _mlp3(gs[None, :], p, 'critic', jnp.tanh)
    return jnp.concatenate([logits[:, 0], value[:, 0]])


# speedup vs baseline: 13.1473x; 1.0217x over previous
import functools

import jax
import jax.numpy as jnp
from jax import lax
from jax.experimental import pallas as pl
from jax.experimental.pallas import tpu as pltpu
from jax.experimental.pallas import tpu_sc as plsc

HID = 8
_NC, _NS = 2, 16


def _sc_agg(res_flat, req_src, req_dst):
    E = req_src.shape[0]
    EPT = E // (_NC * _NS)
    CH = 400
    NROW = CH * 8 // 128
    ACCW = 16384
    mesh = plsc.VectorSubcoreMesh(core_axis_name="c", subcore_axis_name="s")

    @functools.partial(
        pl.kernel, mesh=mesh,
        compiler_params=pltpu.CompilerParams(needs_layout_passes=False),
        out_type=jax.ShapeDtypeStruct((_NC * ACCW,), jnp.float32),
        scratch_types=dict(
            tab=pltpu.VMEM((2000 * 8,), jnp.float32),
            sbuf=pltpu.VMEM((CH,), jnp.int32),
            dbuf=pltpu.VMEM((CH,), jnp.int32),
            idxb=pltpu.VMEM((NROW * 128,), jnp.int32),
            valb=pltpu.VMEM((NROW * 128,), jnp.float32),
            zbuf=pltpu.VMEM((ACCW // _NS,), jnp.float32),
            acc=pltpu.VMEM_SHARED((ACCW,), jnp.float32),
        ),
    )
    def k(res_hbm, src_hbm, dst_hbm, out_hbm, tab, sbuf, dbuf, idxb, valb, zbuf, acc):
        c = lax.axis_index("c")
        s = lax.axis_index("s")
        base = (c * _NS + s) * EPT
        pltpu.sync_copy(res_hbm, tab)
        zero16 = jnp.zeros((16,), jnp.float32)
        zslice = ACCW // _NS

        @pl.loop(0, zslice // 16)
        def _(i):
            zbuf[pl.ds(i * 16, 16)] = zero16

        pltpu.sync_copy(zbuf, acc.at[pl.ds(s * zslice, zslice)])
        plsc.subcore_barrier()
        iota = lax.iota(jnp.int32, 16)

        @pl.loop(0, EPT // CH)
        def _(kk):
            off = base + kk * CH
            pltpu.sync_copy(src_hbm.at[pl.ds(off, CH)], sbuf)
            pltpu.sync_copy(dst_hbm.at[pl.ds(off, CH)], dbuf)

            @pl.loop(0, CH // 16)
            def _(g):
                sv = sbuf[pl.ds(g * 16, 16)]
                dv = dbuf[pl.ds(g * 16, 16)]
                for j in range(8):
                    vals = plsc.load_gather(tab, [dv * 8 + j])
                    pos = g * 128 + iota * 8 + j
                    plsc.store_scatter(valb, [pos], vals)
                    plsc.store_scatter(idxb, [pos], sv * 8 + j)

            pltpu.sync_copy(valb, acc.at[idxb], add=True)

        plsc.subcore_barrier()
        pltpu.sync_copy(acc.at[pl.ds(s * zslice, zslice)],
                        out_hbm.at[pl.ds(c * ACCW + s * zslice, zslice)])

    return k(res_flat, req_src, req_dst)


def _sc_rep_scatter(prec_src, prec_dst, n):
    E = prec_src.shape[0]
    EPT = E // (_NC * _NS)
    CH = 400
    mesh = plsc.VectorSubcoreMesh(core_axis_name="c", subcore_axis_name="s")

    @functools.partial(
        pl.kernel, mesh=mesh,
        compiler_params=pltpu.CompilerParams(needs_layout_passes=False),
        out_type=jax.ShapeDtypeStruct((n * n,), jnp.int32),
        scratch_types=dict(
            sbuf=pltpu.VMEM((CH,), jnp.int32),
            dbuf=pltpu.VMEM((CH,), jnp.int32),
            kbuf=pltpu.VMEM((CH,), jnp.int32),
            ebuf=pltpu.VMEM((CH,), jnp.int32),
        ),
    )
    def k(src_hbm, dst_hbm, out_hbm, sbuf, dbuf, kbuf, ebuf):
        c = lax.axis_index("c")
        s = lax.axis_index("s")
        base = (c * _NS + s) * EPT
        iota = lax.iota(jnp.int32, 16)

        @pl.loop(0, EPT // CH)
        def _(kk):
            off = base + kk * CH
            pltpu.sync_copy(src_hbm.at[pl.ds(off, CH)], sbuf)
            pltpu.sync_copy(dst_hbm.at[pl.ds(off, CH)], dbuf)

            @pl.loop(0, CH // 16)
            def _(g):
                sv = sbuf[pl.ds(g * 16, 16)]
                dv = dbuf[pl.ds(g * 16, 16)]
                kbuf[pl.ds(g * 16, 16)] = sv * n + dv
                ebuf[pl.ds(g * 16, 16)] = off + g * 16 + iota

            pltpu.sync_copy(ebuf, out_hbm.at[kbuf])

    return k(prec_src, prec_dst)


def _sc_prec(ops_flat, prec_src, prec_dst, rep_tab, n, f):
    E = prec_src.shape[0]
    EPT = E // (_NC * _NS)
    CH = 400
    NP = 10240
    NP8 = 81920
    SEG = 2 * NP8 + 2 * NP
    mesh = plsc.VectorSubcoreMesh(core_axis_name="c", subcore_axis_name="s")

    @functools.partial(
        pl.kernel, mesh=mesh,
        compiler_params=pltpu.CompilerParams(needs_layout_passes=False),
        out_type=jax.ShapeDtypeStruct((_NC * SEG,), jnp.float32),
        scratch_types=dict(
            tab=pltpu.VMEM((n * f,), jnp.float32),
            sbuf=pltpu.VMEM((CH,), jnp.int32),
            dbuf=pltpu.VMEM((CH,), jnp.int32),
            kbuf=pltpu.VMEM((CH,), jnp.int32),
            rbuf=pltpu.VMEM((CH,), jnp.int32),
            iv=pltpu.VMEM((CH * f,), jnp.int32),
            vv=pltpu.VMEM((CH * f,), jnp.float32),
            ov=pltpu.VMEM((CH * f,), jnp.int32),
            wv=pltpu.VMEM((CH * f,), jnp.float32),
            ic=pltpu.VMEM((2 * CH,), jnp.int32),
            vc=pltpu.VMEM((2 * CH,), jnp.float32),
            zbuf=pltpu.VMEM((SEG // _NS,), jnp.float32),
            acc=pltpu.VMEM_SHARED((SEG,), jnp.float32),
        ),
    )
    def k(ops_hbm, src_hbm, dst_hbm, rep_hbm, out_hbm,
          tab, sbuf, dbuf, kbuf, rbuf, iv, vv, ov, wv, ic, vc, zbuf, acc):
        c = lax.axis_index("c")
        s = lax.axis_index("s")
        base = (c * _NS + s) * EPT
        pltpu.sync_copy(ops_hbm, tab)
        zero16 = jnp.zeros((16,), jnp.float32)
        zslice = SEG // _NS

        @pl.loop(0, zslice // 16)
        def _(i):
            zbuf[pl.ds(i * 16, 16)] = zero16

        pltpu.sync_copy(zbuf, acc.at[pl.ds(s * zslice, zslice)])
        plsc.subcore_barrier()
        iota = lax.iota(jnp.int32, 16)

        @pl.loop(0, EPT // CH)
        def _(kk):
            off = base + kk * CH
            pltpu.sync_copy(src_hbm.at[pl.ds(off, CH)], sbuf)
            pltpu.sync_copy(dst_hbm.at[pl.ds(off, CH)], dbuf)

            @pl.loop(0, CH // 16)
            def _(g):
                sv = sbuf[pl.ds(g * 16, 16)]
                dv = dbuf[pl.ds(g * 16, 16)]
                kbuf[pl.ds(g * 16, 16)] = sv * n + dv

            pltpu.sync_copy(rep_hbm.at[kbuf], rbuf)

            @pl.loop(0, CH // 16)
            def _(g):
                sv = sbuf[pl.ds(g * 16, 16)]
                dv = dbuf[pl.ds(g * 16, 16)]
                rv = rbuf[pl.ds(g * 16, 16)]
                eid = off + g * 16 + iota
                w = jnp.where(rv == eid, 1.0, 0.0).astype(jnp.float32)
                for j in range(f):
                    pos = g * 16 * f + iota * f + j
                    vs = plsc.load_gather(tab, [sv * f + j]) * w
                    plsc.store_scatter(wv, [pos], vs)
                    plsc.store_scatter(iv, [pos], dv * f + j)
                    vd = plsc.load_gather(tab, [dv * f + j]) * w
                    plsc.store_scatter(vv, [pos], vd)
                    plsc.store_scatter(ov, [pos], NP8 + sv * f + j)
                cpos = g * 32 + iota * 2
                plsc.store_scatter(vc, [cpos], w)
                plsc.store_scatter(ic, [cpos], 2 * NP8 + dv)
                plsc.store_scatter(vc, [cpos + 1], w)
                plsc.store_scatter(ic, [cpos + 1], 2 * NP8 + NP + sv)

            pltpu.sync_copy(wv, acc.at[iv], add=True)
            pltpu.sync_copy(vv, acc.at[ov], add=True)
            pltpu.sync_copy(vc, acc.at[ic], add=True)

        plsc.subcore_barrier()
        pltpu.sync_copy(acc.at[pl.ds(s * zslice, zslice)],
                        out_hbm.at[pl.ds(c * SEG + s * zslice, zslice)])

    return k(ops_flat, prec_src, prec_dst, rep_tab)


def _sc_res_edges(uc, vc, o_flat, req_src, req_dst):
    E = req_src.shape[0]
    EPT = E // (_NC * _NS)
    CH = 400
    ACCW = 16384
    SEGR = ACCW + 256 + 128
    mesh = plsc.VectorSubcoreMesh(core_axis_name="c", subcore_axis_name="s")

    @functools.partial(
        pl.kernel, mesh=mesh,
        compiler_params=pltpu.CompilerParams(needs_layout_passes=False),
        out_type=jax.ShapeDtypeStruct((_NC * SEGR,), jnp.float32),
        scratch_types=dict(
            utab=pltpu.VMEM((2000,), jnp.float32),
            vtab=pltpu.VMEM((2000,), jnp.float32),
            otab=pltpu.VMEM((16000,), jnp.float32),
            sbuf=pltpu.VMEM((CH,), jnp.int32),
            dbuf=pltpu.VMEM((CH,), jnp.int32),
            idxb=pltpu.VMEM((CH * 8,), jnp.int32),
            valb=pltpu.VMEM((CH * 8,), jnp.float32),
            mbuf=pltpu.VMEM((16 * _NS,), jnp.float32),
            zbuf=pltpu.VMEM((ACCW // _NS,), jnp.float32),
            acc=pltpu.VMEM_SHARED((ACCW,), jnp.float32),
            mtab=pltpu.VMEM_SHARED((16 * _NS,), jnp.float32),
            ztab=pltpu.VMEM_SHARED((16 * _NS,), jnp.float32),
        ),
    )
    def k(uc_hbm, vc_hbm, o_hbm, src_hbm, dst_hbm, out_hbm,
          utab, vtab, otab, sbuf, dbuf, idxb, valb, mbuf, zbuf,
          acc, mtab, ztab):
        c = lax.axis_index("c")
        s = lax.axis_index("s")
        base = (c * _NS + s) * EPT
        pltpu.sync_copy(uc_hbm, utab)
        pltpu.sync_copy(vc_hbm, vtab)
        pltpu.sync_copy(o_hbm, otab)
        zero16 = jnp.zeros((16,), jnp.float32)
        zslice = ACCW // _NS

        @pl.loop(0, zslice // 16)
        def _(i):
            zbuf[pl.ds(i * 16, 16)] = zero16

        pltpu.sync_copy(zbuf, acc.at[pl.ds(s * zslice, zslice)])
        iota = lax.iota(jnp.int32, 16)
        neg = jnp.full((16,), -3.0e38, jnp.float32)

        def ca_of(g):
            sv = sbuf[pl.ds(g * 16, 16)]
            dv = dbuf[pl.ds(g * 16, 16)]
            x = plsc.load_gather(utab, [dv]) + plsc.load_gather(vtab, [sv])
            return jnp.where(x >= 0.0, x, 0.2 * x)

        def p1_chunk(kk, mv):
            off = base + kk * CH
            pltpu.sync_copy(src_hbm.at[pl.ds(off, CH)], sbuf)
            pltpu.sync_copy(dst_hbm.at[pl.ds(off, CH)], dbuf)

            def p1_g(g, mv2):
                return jnp.maximum(mv2, ca_of(g))

            return pl.loop(0, CH // 16, init_carry=mv)(p1_g)

        mv = pl.loop(0, EPT // CH, init_carry=neg)(p1_chunk)
        mbuf[pl.ds(0, 16)] = mv
        pltpu.sync_copy(mbuf.at[pl.ds(0, 16)], mtab.at[pl.ds(s * 16, 16)])
        plsc.subcore_barrier()
        pltpu.sync_copy(mtab, mbuf)

        def mred(kidx, mv2):
            return jnp.maximum(mv2, mbuf[pl.ds(kidx * 16, 16)])

        mv = pl.loop(0, _NS, init_carry=neg)(mred)
        m = lax.reduce_max(mv, axes=(0,))

        def p2_chunk(kk, zv):
            off = base + kk * CH
            pltpu.sync_copy(src_hbm.at[pl.ds(off, CH)], sbuf)
            pltpu.sync_copy(dst_hbm.at[pl.ds(off, CH)], dbuf)

            def p2_g(g, zv2):
                sv = sbuf[pl.ds(g * 16, 16)]
                dv = dbuf[pl.ds(g * 16, 16)]
                t = jnp.exp(ca_of(g) - m)
                for j in range(8):
                    pos = g * 128 + iota * 8 + j
                    vals = plsc.load_gather(otab, [sv * 8 + j]) * t
                    plsc.store_scatter(valb, [pos], vals)
                    plsc.store_scatter(idxb, [pos], dv * 8 + j)
                return zv2 + t

            zv = pl.loop(0, CH // 16, init_carry=zv)(p2_g)
            pltpu.sync_copy(valb, acc.at[idxb], add=True)
            return zv

        zv = pl.loop(0, EPT // CH, init_carry=zero16)(p2_chunk)
        mbuf[pl.ds(0, 16)] = zv
        pltpu.sync_copy(mbuf.at[pl.ds(0, 16)], ztab.at[pl.ds(s * 16, 16)])
        plsc.subcore_barrier()
        obase = c * SEGR
        pltpu.sync_copy(acc.at[pl.ds(s * zslice, zslice)],
                        out_hbm.at[pl.ds(obase + s * zslice, zslice)])

        @pl.when(s == 0)
        def _():
            pltpu.sync_copy(ztab, out_hbm.at[pl.ds(obase + ACCW, 256)])
            mbuf[pl.ds(0, 16)] = jnp.full((16,), m, jnp.float32)
            pltpu.sync_copy(mbuf.at[pl.ds(0, 16)],
                            out_hbm.at[pl.ds(obase + ACCW + 256, 16)])

    return k(uc, vc, o_flat, req_src, req_dst)


def _mlp3(x, p, pre, act):
    h = act(x @ p[pre + '_w0'] + p[pre + '_b0'])
    h = act(h @ p[pre + '_w1'] + p[pre + '_b1'])
    return h @ p[pre + '_w2'] + p[pre + '_b2']


_MLP_TENSORS = ('_w0', '_b0', '_w1', '_b1', '_w2', '_b2')


def _wlist(p, pre, names):
    ws = []
    for nm in names:
        for t in _MLP_TENSORS:
            a = p[pre + nm + t]
            ws.append(a.reshape(1, -1) if a.ndim == 1 else a)
    return ws


def _mlp_refs(x, wrefs, act):
    w0, b0, w1, b1, w2, b2 = (r[...] for r in wrefs)
    h = act(x @ w0 + b0)
    h = act(h @ w1 + b1)
    return h @ w2 + b2


def _tc_op_mlps(in_sum, in_cnt, out_sum, out_cnt, ops, agg, p, pre):
    N, F = ops.shape
    B = 1000
    ws = _wlist(p, pre, ['pred', 'succ', 'same', 'res', 'comb'])

    def elu(x):
        return jnp.where(x > 0, x, jnp.exp(jnp.minimum(x, 0.0)) - 1.0)

    def body(*refs):
        is_ref, ic_ref, os_ref, oc_ref, ops_ref, agg_ref = refs[:6]
        wrefs = refs[6:36]
        out_ref = refs[36]
        i = pl.program_id(0)
        pm = is_ref[...] / ic_ref[...]
        sm = os_ref[...] / oc_ref[...]
        preds = _mlp_refs(pm, wrefs[0:6], elu)
        succs = _mlp_refs(sm, wrefs[6:12], elu)
        same = _mlp_refs(ops_ref[...], wrefs[12:18], elu)
        aggm = _mlp_refs(agg_ref[...], wrefs[18:24], elu)
        comb = _mlp_refs(jnp.concatenate([preds, succs, aggm, same], -1),
                         wrefs[24:30], elu)
        rid = i * B + lax.broadcasted_iota(jnp.int32, (B, 1), 0)
        mask = (rid > 0) & (rid < N - 1)
        out_ref[...] = jnp.where(mask, comb, 0.0)

    wspecs = [pl.BlockSpec(w.shape, lambda i: (0, 0)) for w in ws]
    return pl.pallas_call(
        body,
        grid=(N // B,),
        in_specs=[
            pl.BlockSpec((B, F), lambda i: (i, 0)),
            pl.BlockSpec((B, 1), lambda i: (i, 0)),
            pl.BlockSpec((B, F), lambda i: (i, 0)),
            pl.BlockSpec((B, 1), lambda i: (i, 0)),
            pl.BlockSpec((B, F), lambda i: (i, 0)),
            pl.BlockSpec((B, 8), lambda i: (i, 0)),
        ] + wspecs,
        out_specs=pl.BlockSpec((B, HID), lambda i: (i, 0)),
        out_shape=jax.ShapeDtypeStruct((N, HID), jnp.float32),
    )(in_sum, in_cnt[:, None], out_sum, out_cnt[:, None], ops, agg, *ws)


def _sc_action_gather(ops2000_flat, res_flat, actions_flat):
    mesh = plsc.VectorSubcoreMesh(core_axis_name="c", subcore_axis_name="s")

    @functools.partial(
        pl.kernel, mesh=mesh,
        compiler_params=pltpu.CompilerParams(needs_layout_passes=False),
        out_type=jax.ShapeDtypeStruct((4096,), jnp.float32),
        scratch_types=dict(
            tab=pltpu.VMEM((32000,), jnp.float32),
            atab=pltpu.VMEM((512,), jnp.int32),
            obuf=pltpu.VMEM((128,), jnp.float32),
        ),
    )
    def k(ops_hbm, res_hbm, act_hbm, out_hbm, tab, atab, obuf):
        c = lax.axis_index("c")
        s = lax.axis_index("s")
        tile = c * _NS + s
        pltpu.sync_copy(ops_hbm, tab.at[pl.ds(0, 16000)])
        pltpu.sync_copy(res_hbm, tab.at[pl.ds(16000, 16000)])
        pltpu.sync_copy(act_hbm, atab)
        iota = lax.iota(jnp.int32, 16)

        @pl.loop(0, 8)
        def _(g):
            i = tile * 8 + g
            av = plsc.load_gather(atab, [jnp.full((16,), 0, jnp.int32) + 2 * i])
            bv = plsc.load_gather(atab, [jnp.full((16,), 1, jnp.int32) + 2 * i])
            src = jnp.where(iota < 8, av * 8 + iota, 16000 + bv * 8 + iota - 8)
            obuf[pl.ds(g * 16, 16)] = plsc.load_gather(tab, [src])

        pltpu.sync_copy(obuf, out_hbm.at[pl.ds(tile * 128, 128)])

    return k(ops2000_flat, res_flat, actions_flat)


def _tc_head(ops, res, feat16, p):
    wa = _wlist(p, 'actor', [''])
    wc = _wlist(p, 'critic', [''])
    A = feat16.shape[0]

    def body(ops_ref, res_ref, f_ref, *rest):
        warefs = rest[0:6]
        wcrefs = rest[6:12]
        out_ref = rest[12]
        gs_o = jnp.mean(ops_ref[...], axis=0)
        gs_r = jnp.mean(res_ref[...], axis=0)
        gs = jnp.concatenate([gs_o, gs_r])[None, :]
        feat = jnp.concatenate(
            [f_ref[...], jnp.broadcast_to(gs, (A, 16))], -1)
        logits = _mlp_refs(feat, warefs, jnp.tanh)
        value = _mlp_refs(gs, wcrefs, jnp.tanh)
        out_ref[...] = jnp.concatenate(
            [logits[:, 0], value[:, 0]])

    return pl.pallas_call(
        body,
        in_specs=[pl.BlockSpec(ops.shape, lambda: (0, 0)),
                  pl.BlockSpec(res.shape, lambda: (0, 0)),
                  pl.BlockSpec(feat16.shape, lambda: (0, 0))]
        + [pl.BlockSpec(w.shape, lambda: (0, 0)) for w in wa + wc],
        out_specs=pl.BlockSpec((A + 1,), lambda: (0,)),
        out_shape=jax.ShapeDtypeStruct((A + 1,), jnp.float32),
    )(ops, res, feat16, *(wa + wc))


def _res_layer(resources, operations, req, p, pre):
    r = resources @ p[pre + 'Wr']
    o2000 = operations[:2000] @ p[pre + 'Wo']
    ac = p[pre + 'a_cross']
    uc = (r @ ac[:8]).reshape(-1)
    vc = (o2000 @ ac[8:]).reshape(-1)
    sa = jax.nn.leaky_relu(jnp.concatenate([r, r], -1) @ p[pre + 'a_self'], 0.2)
    out = _sc_res_edges(uc, vc, o2000.reshape(-1), req[0], req[1])
    SEGR = 16384 + 256 + 128
    o0, o1 = out[:SEGR], out[SEGR:]
    acc0 = o0[:16000].reshape(2000, 8)
    acc1 = o1[:16000].reshape(2000, 8)
    z0 = o0[16384:16640].sum()
    z1 = o1[16384:16640].sum()
    m0 = o0[16640]
    m1 = o1[16640]
    m = jnp.maximum(jnp.maximum(m0, m1), sa.max())
    e0 = jnp.exp(m0 - m)
    e1 = jnp.exp(m1 - m)
    esa = jnp.exp(sa - m)
    Z = z0 * e0 + z1 * e1 + esa.sum()
    summed = (acc0 * e0 + acc1 * e1) / Z
    ns = esa / Z
    return jax.nn.elu(ns * r + summed)


def _op_layer(operations, resources, prec, req, rep, p, pre):
    N, F = operations.shape
    parts = _sc_agg(resources.reshape(-1), req[0], req[1])
    agg2000 = (parts[:16000] + parts[16384:16384 + 16000]).reshape(2000, 8)
    agg = jnp.concatenate(
        [agg2000, jnp.zeros((N - 2000, 8), operations.dtype)], axis=0)
    NP, NP8 = 10240, 81920
    SEG = 2 * NP8 + 2 * NP
    pp = _sc_prec(operations.reshape(-1), prec[0], prec[1], rep, N, F)
    tot = pp[:SEG] + pp[SEG:]
    in_sum = tot[0:N * F].reshape(N, F)
    out_sum = tot[NP8:NP8 + N * F].reshape(N, F)
    in_cnt = tot[2 * NP8:2 * NP8 + N]
    out_cnt = tot[2 * NP8 + NP:2 * NP8 + NP + N]
    return _tc_op_mlps(in_sum, in_cnt, out_sum, out_cnt, operations, agg, p, pre)


def kernel(operations, resources, precedence_edges, requirement_edges, actions, params):
    p = params
    prec, req = precedence_edges, requirement_edges
    N = operations.shape[0]
    rep = _sc_rep_scatter(prec[0], prec[1], N)

    ops, res = operations, resources
    for l in range(2):
        res = _res_layer(res, ops, req, p, 'r%d_' % l)
        ops = _op_layer(ops, res, prec, req, rep, p, 'o%d_' % l)
    feat16 = _sc_action_gather(ops[:2000].reshape(-1), res.reshape(-1),
                               actions.reshape(-1)).reshape(256, 16)
    return _tc_head(ops, res, feat16, p)


# speedup vs baseline: 15.1160x; 1.1497x over previous
import functools

import jax
import jax.numpy as jnp
from jax import lax
from jax.experimental import pallas as pl
from jax.experimental.pallas import tpu as pltpu
from jax.experimental.pallas import tpu_sc as plsc

HID = 8
_NC, _NS = 2, 16


def _sc_agg(res_flat, req_src, req_dst):
    E = req_src.shape[0]
    EPT = E // (_NC * _NS)
    CH = 2000
    NROW = CH * 8 // 128
    ACCW = 16384
    mesh = plsc.VectorSubcoreMesh(core_axis_name="c", subcore_axis_name="s")

    @functools.partial(
        pl.kernel, mesh=mesh,
        compiler_params=pltpu.CompilerParams(needs_layout_passes=False),
        out_type=jax.ShapeDtypeStruct((_NC * ACCW,), jnp.float32),
        scratch_types=dict(
            tab=pltpu.VMEM((2000 * 8,), jnp.float32),
            sbuf=pltpu.VMEM((CH,), jnp.int32),
            dbuf=pltpu.VMEM((CH,), jnp.int32),
            idxb=pltpu.VMEM((NROW * 128,), jnp.int32),
            valb=pltpu.VMEM((NROW * 128,), jnp.float32),
            zbuf=pltpu.VMEM((ACCW // _NS,), jnp.float32),
            acc=pltpu.VMEM_SHARED((ACCW,), jnp.float32),
        ),
    )
    def k(res_hbm, src_hbm, dst_hbm, out_hbm, tab, sbuf, dbuf, idxb, valb, zbuf, acc):
        c = lax.axis_index("c")
        s = lax.axis_index("s")
        base = (c * _NS + s) * EPT
        pltpu.sync_copy(res_hbm, tab)
        zero16 = jnp.zeros((16,), jnp.float32)
        zslice = ACCW // _NS

        @pl.loop(0, zslice // 16)
        def _(i):
            zbuf[pl.ds(i * 16, 16)] = zero16

        pltpu.sync_copy(zbuf, acc.at[pl.ds(s * zslice, zslice)])
        plsc.subcore_barrier()
        iota = lax.iota(jnp.int32, 16)

        @pl.loop(0, EPT // CH)
        def _(kk):
            off = base + kk * CH
            pltpu.sync_copy(src_hbm.at[pl.ds(off, CH)], sbuf)
            pltpu.sync_copy(dst_hbm.at[pl.ds(off, CH)], dbuf)

            @pl.loop(0, CH // 16)
            def _(g):
                sv = sbuf[pl.ds(g * 16, 16)]
                dv = dbuf[pl.ds(g * 16, 16)]
                for j in range(8):
                    vals = plsc.load_gather(tab, [dv * 8 + j])
                    pos = g * 128 + iota * 8 + j
                    plsc.store_scatter(valb, [pos], vals)
                    plsc.store_scatter(idxb, [pos], sv * 8 + j)

            pltpu.sync_copy(valb, acc.at[idxb], add=True)

        plsc.subcore_barrier()
        pltpu.sync_copy(acc.at[pl.ds(s * zslice, zslice)],
                        out_hbm.at[pl.ds(c * ACCW + s * zslice, zslice)])

    return k(res_flat, req_src, req_dst)


def _sc_rep_scatter(prec_src, prec_dst, n):
    E = prec_src.shape[0]
    EPT = E // (_NC * _NS)
    CH = 4000
    mesh = plsc.VectorSubcoreMesh(core_axis_name="c", subcore_axis_name="s")

    @functools.partial(
        pl.kernel, mesh=mesh,
        compiler_params=pltpu.CompilerParams(needs_layout_passes=False),
        out_type=jax.ShapeDtypeStruct((n * n,), jnp.int32),
        scratch_types=dict(
            sbuf=pltpu.VMEM((CH,), jnp.int32),
            dbuf=pltpu.VMEM((CH,), jnp.int32),
            kbuf=pltpu.VMEM((CH,), jnp.int32),
            ebuf=pltpu.VMEM((CH,), jnp.int32),
        ),
    )
    def k(src_hbm, dst_hbm, out_hbm, sbuf, dbuf, kbuf, ebuf):
        c = lax.axis_index("c")
        s = lax.axis_index("s")
        base = (c * _NS + s) * EPT
        iota = lax.iota(jnp.int32, 16)

        @pl.loop(0, EPT // CH)
        def _(kk):
            off = base + kk * CH
            pltpu.sync_copy(src_hbm.at[pl.ds(off, CH)], sbuf)
            pltpu.sync_copy(dst_hbm.at[pl.ds(off, CH)], dbuf)

            @pl.loop(0, CH // 16)
            def _(g):
                sv = sbuf[pl.ds(g * 16, 16)]
                dv = dbuf[pl.ds(g * 16, 16)]
                kbuf[pl.ds(g * 16, 16)] = sv * n + dv
                ebuf[pl.ds(g * 16, 16)] = off + g * 16 + iota

            pltpu.sync_copy(ebuf, out_hbm.at[kbuf])

    return k(prec_src, prec_dst)


def _sc_prec(ops_flat, prec_src, prec_dst, rep_tab, n, f):
    E = prec_src.shape[0]
    EPT = E // (_NC * _NS)
    CH = 800
    NP = 10240
    NP8 = 81920
    SEG = 2 * NP8 + 2 * NP
    mesh = plsc.VectorSubcoreMesh(core_axis_name="c", subcore_axis_name="s")

    @functools.partial(
        pl.kernel, mesh=mesh,
        compiler_params=pltpu.CompilerParams(needs_layout_passes=False),
        out_type=jax.ShapeDtypeStruct((_NC * SEG,), jnp.float32),
        scratch_types=dict(
            tab=pltpu.VMEM((n * f,), jnp.float32),
            sbuf=pltpu.VMEM((CH,), jnp.int32),
            dbuf=pltpu.VMEM((CH,), jnp.int32),
            kbuf=pltpu.VMEM((CH,), jnp.int32),
            rbuf=pltpu.VMEM((CH,), jnp.int32),
            iv=pltpu.VMEM((CH * f,), jnp.int32),
            vv=pltpu.VMEM((CH * f,), jnp.float32),
            ov=pltpu.VMEM((CH * f,), jnp.int32),
            wv=pltpu.VMEM((CH * f,), jnp.float32),
            ic=pltpu.VMEM((2 * CH,), jnp.int32),
            vc=pltpu.VMEM((2 * CH,), jnp.float32),
            zbuf=pltpu.VMEM((1920,), jnp.float32),
            acc=pltpu.VMEM_SHARED((SEG,), jnp.float32),
        ),
    )
    def k(ops_hbm, src_hbm, dst_hbm, rep_hbm, out_hbm,
          tab, sbuf, dbuf, kbuf, rbuf, iv, vv, ov, wv, ic, vc, zbuf, acc):
        c = lax.axis_index("c")
        s = lax.axis_index("s")
        base = (c * _NS + s) * EPT
        pltpu.sync_copy(ops_hbm, tab)
        zero16 = jnp.zeros((16,), jnp.float32)
        zslice = SEG // _NS

        @pl.loop(0, 120)
        def _(i):
            zbuf[pl.ds(i * 16, 16)] = zero16

        @pl.loop(0, 6)
        def _(q):
            pltpu.sync_copy(zbuf, acc.at[pl.ds(s * zslice + q * 1920, 1920)])

        plsc.subcore_barrier()
        iota = lax.iota(jnp.int32, 16)

        @pl.loop(0, EPT // CH)
        def _(kk):
            off = base + kk * CH
            pltpu.sync_copy(src_hbm.at[pl.ds(off, CH)], sbuf)
            pltpu.sync_copy(dst_hbm.at[pl.ds(off, CH)], dbuf)

            @pl.loop(0, CH // 16)
            def _(g):
                sv = sbuf[pl.ds(g * 16, 16)]
                dv = dbuf[pl.ds(g * 16, 16)]
                kbuf[pl.ds(g * 16, 16)] = sv * n + dv

            pltpu.sync_copy(rep_hbm.at[kbuf], rbuf)

            @pl.loop(0, CH // 16)
            def _(g):
                sv = sbuf[pl.ds(g * 16, 16)]
                dv = dbuf[pl.ds(g * 16, 16)]
                rv = rbuf[pl.ds(g * 16, 16)]
                eid = off + g * 16 + iota
                w = jnp.where(rv == eid, 1.0, 0.0).astype(jnp.float32)
                for j in range(f):
                    pos = g * 16 * f + iota * f + j
                    vs = plsc.load_gather(tab, [sv * f + j]) * w
                    plsc.store_scatter(wv, [pos], vs)
                    plsc.store_scatter(iv, [pos], dv * f + j)
                    vd = plsc.load_gather(tab, [dv * f + j]) * w
                    plsc.store_scatter(vv, [pos], vd)
                    plsc.store_scatter(ov, [pos], NP8 + sv * f + j)
                cpos = g * 32 + iota * 2
                plsc.store_scatter(vc, [cpos], w)
                plsc.store_scatter(ic, [cpos], 2 * NP8 + dv)
                plsc.store_scatter(vc, [cpos + 1], w)
                plsc.store_scatter(ic, [cpos + 1], 2 * NP8 + NP + sv)

            pltpu.sync_copy(wv, acc.at[iv], add=True)
            pltpu.sync_copy(vv, acc.at[ov], add=True)
            pltpu.sync_copy(vc, acc.at[ic], add=True)

        plsc.subcore_barrier()
        pltpu.sync_copy(acc.at[pl.ds(s * zslice, zslice)],
                        out_hbm.at[pl.ds(c * SEG + s * zslice, zslice)])

    return k(ops_flat, prec_src, prec_dst, rep_tab)


def _sc_res_edges(uc, vc, o_flat, req_src, req_dst):
    E = req_src.shape[0]
    EPT = E // (_NC * _NS)
    CH = 2000
    ACCW = 16384
    SEGR = ACCW + 256 + 128
    mesh = plsc.VectorSubcoreMesh(core_axis_name="c", subcore_axis_name="s")

    @functools.partial(
        pl.kernel, mesh=mesh,
        compiler_params=pltpu.CompilerParams(needs_layout_passes=False),
        out_type=jax.ShapeDtypeStruct((_NC * SEGR,), jnp.float32),
        scratch_types=dict(
            utab=pltpu.VMEM((2000,), jnp.float32),
            vtab=pltpu.VMEM((2000,), jnp.float32),
            otab=pltpu.VMEM((16000,), jnp.float32),
            sbuf=pltpu.VMEM((CH,), jnp.int32),
            dbuf=pltpu.VMEM((CH,), jnp.int32),
            idxb=pltpu.VMEM((CH * 8,), jnp.int32),
            valb=pltpu.VMEM((CH * 8,), jnp.float32),
            mbuf=pltpu.VMEM((16 * _NS,), jnp.float32),
            zbuf=pltpu.VMEM((ACCW // _NS,), jnp.float32),
            acc=pltpu.VMEM_SHARED((ACCW,), jnp.float32),
            mtab=pltpu.VMEM_SHARED((16 * _NS,), jnp.float32),
            ztab=pltpu.VMEM_SHARED((16 * _NS,), jnp.float32),
        ),
    )
    def k(uc_hbm, vc_hbm, o_hbm, src_hbm, dst_hbm, out_hbm,
          utab, vtab, otab, sbuf, dbuf, idxb, valb, mbuf, zbuf,
          acc, mtab, ztab):
        c = lax.axis_index("c")
        s = lax.axis_index("s")
        base = (c * _NS + s) * EPT
        pltpu.sync_copy(uc_hbm, utab)
        pltpu.sync_copy(vc_hbm, vtab)
        pltpu.sync_copy(o_hbm, otab)
        zero16 = jnp.zeros((16,), jnp.float32)
        zslice = ACCW // _NS

        @pl.loop(0, zslice // 16)
        def _(i):
            zbuf[pl.ds(i * 16, 16)] = zero16

        pltpu.sync_copy(zbuf, acc.at[pl.ds(s * zslice, zslice)])
        iota = lax.iota(jnp.int32, 16)
        neg = jnp.full((16,), -3.0e38, jnp.float32)

        def ca_of(g):
            sv = sbuf[pl.ds(g * 16, 16)]
            dv = dbuf[pl.ds(g * 16, 16)]
            x = plsc.load_gather(utab, [dv]) + plsc.load_gather(vtab, [sv])
            return jnp.where(x >= 0.0, x, 0.2 * x)

        def p1_chunk(kk, mv):
            off = base + kk * CH
            pltpu.sync_copy(src_hbm.at[pl.ds(off, CH)], sbuf)
            pltpu.sync_copy(dst_hbm.at[pl.ds(off, CH)], dbuf)

            def p1_g(g, mv2):
                return jnp.maximum(mv2, ca_of(g))

            return pl.loop(0, CH // 16, init_carry=mv)(p1_g)

        mv = pl.loop(0, EPT // CH, init_carry=neg)(p1_chunk)
        mbuf[pl.ds(0, 16)] = mv
        pltpu.sync_copy(mbuf.at[pl.ds(0, 16)], mtab.at[pl.ds(s * 16, 16)])
        plsc.subcore_barrier()
        pltpu.sync_copy(mtab, mbuf)

        def mred(kidx, mv2):
            return jnp.maximum(mv2, mbuf[pl.ds(kidx * 16, 16)])

        mv = pl.loop(0, _NS, init_carry=neg)(mred)
        m = lax.reduce_max(mv, axes=(0,))

        def p2_chunk(kk, zv):
            off = base + kk * CH
            pltpu.sync_copy(src_hbm.at[pl.ds(off, CH)], sbuf)
            pltpu.sync_copy(dst_hbm.at[pl.ds(off, CH)], dbuf)

            def p2_g(g, zv2):
                sv = sbuf[pl.ds(g * 16, 16)]
                dv = dbuf[pl.ds(g * 16, 16)]
                t = jnp.exp(ca_of(g) - m)
                for j in range(8):
                    pos = g * 128 + iota * 8 + j
                    vals = plsc.load_gather(otab, [sv * 8 + j]) * t
                    plsc.store_scatter(valb, [pos], vals)
                    plsc.store_scatter(idxb, [pos], dv * 8 + j)
                return zv2 + t

            zv = pl.loop(0, CH // 16, init_carry=zv)(p2_g)
            pltpu.sync_copy(valb, acc.at[idxb], add=True)
            return zv

        zv = pl.loop(0, EPT // CH, init_carry=zero16)(p2_chunk)
        mbuf[pl.ds(0, 16)] = zv
        pltpu.sync_copy(mbuf.at[pl.ds(0, 16)], ztab.at[pl.ds(s * 16, 16)])
        plsc.subcore_barrier()
        obase = c * SEGR
        pltpu.sync_copy(acc.at[pl.ds(s * zslice, zslice)],
                        out_hbm.at[pl.ds(obase + s * zslice, zslice)])

        @pl.when(s == 0)
        def _():
            pltpu.sync_copy(ztab, out_hbm.at[pl.ds(obase + ACCW, 256)])
            mbuf[pl.ds(0, 16)] = jnp.full((16,), m, jnp.float32)
            pltpu.sync_copy(mbuf.at[pl.ds(0, 16)],
                            out_hbm.at[pl.ds(obase + ACCW + 256, 16)])

    return k(uc, vc, o_flat, req_src, req_dst)


def _mlp3(x, p, pre, act):
    h = act(x @ p[pre + '_w0'] + p[pre + '_b0'])
    h = act(h @ p[pre + '_w1'] + p[pre + '_b1'])
    return h @ p[pre + '_w2'] + p[pre + '_b2']


_MLP_TENSORS = ('_w0', '_b0', '_w1', '_b1', '_w2', '_b2')


def _wlist(p, pre, names):
    ws = []
    for nm in names:
        for t in _MLP_TENSORS:
            a = p[pre + nm + t]
            ws.append(a.reshape(1, -1) if a.ndim == 1 else a)
    return ws


def _mlp_refs(x, wrefs, act):
    w0, b0, w1, b1, w2, b2 = (r[...] for r in wrefs)
    h = act(x @ w0 + b0)
    h = act(h @ w1 + b1)
    return h @ w2 + b2


def _tc_op_mlps(in_sum, in_cnt, out_sum, out_cnt, ops, agg, p, pre):
    N, F = ops.shape
    B = 1000
    ws = _wlist(p, pre, ['pred', 'succ', 'same', 'res', 'comb'])

    def elu(x):
        return jnp.where(x > 0, x, jnp.exp(jnp.minimum(x, 0.0)) - 1.0)

    def body(*refs):
        is_ref, ic_ref, os_ref, oc_ref, ops_ref, agg_ref = refs[:6]
        wrefs = refs[6:36]
        out_ref = refs[36]
        i = pl.program_id(0)
        pm = is_ref[...] / ic_ref[...]
        sm = os_ref[...] / oc_ref[...]
        preds = _mlp_refs(pm, wrefs[0:6], elu)
        succs = _mlp_refs(sm, wrefs[6:12], elu)
        same = _mlp_refs(ops_ref[...], wrefs[12:18], elu)
        aggm = _mlp_refs(agg_ref[...], wrefs[18:24], elu)
        comb = _mlp_refs(jnp.concatenate([preds, succs, aggm, same], -1),
                         wrefs[24:30], elu)
        rid = i * B + lax.broadcasted_iota(jnp.int32, (B, 1), 0)
        mask = (rid > 0) & (rid < N - 1)
        out_ref[...] = jnp.where(mask, comb, 0.0)

    wspecs = [pl.BlockSpec(w.shape, lambda i: (0, 0)) for w in ws]
    return pl.pallas_call(
        body,
        grid=(N // B,),
        in_specs=[
            pl.BlockSpec((B, F), lambda i: (i, 0)),
            pl.BlockSpec((B, 1), lambda i: (i, 0)),
            pl.BlockSpec((B, F), lambda i: (i, 0)),
            pl.BlockSpec((B, 1), lambda i: (i, 0)),
            pl.BlockSpec((B, F), lambda i: (i, 0)),
            pl.BlockSpec((B, 8), lambda i: (i, 0)),
        ] + wspecs,
        out_specs=pl.BlockSpec((B, HID), lambda i: (i, 0)),
        out_shape=jax.ShapeDtypeStruct((N, HID), jnp.float32),
    )(in_sum, in_cnt[:, None], out_sum, out_cnt[:, None], ops, agg, *ws)


def _sc_action_gather(ops2000_flat, res_flat, actions_flat):
    mesh = plsc.VectorSubcoreMesh(core_axis_name="c", subcore_axis_name="s")

    @functools.partial(
        pl.kernel, mesh=mesh,
        compiler_params=pltpu.CompilerParams(needs_layout_passes=False),
        out_type=jax.ShapeDtypeStruct((4096,), jnp.float32),
        scratch_types=dict(
            tab=pltpu.VMEM((32000,), jnp.float32),
            atab=pltpu.VMEM((512,), jnp.int32),
            obuf=pltpu.VMEM((128,), jnp.float32),
        ),
    )
    def k(ops_hbm, res_hbm, act_hbm, out_hbm, tab, atab, obuf):
        c = lax.axis_index("c")
        s = lax.axis_index("s")
        tile = c * _NS + s
        pltpu.sync_copy(ops_hbm, tab.at[pl.ds(0, 16000)])
        pltpu.sync_copy(res_hbm, tab.at[pl.ds(16000, 16000)])
        pltpu.sync_copy(act_hbm, atab)
        iota = lax.iota(jnp.int32, 16)

        @pl.loop(0, 8)
        def _(g):
            i = tile * 8 + g
            av = plsc.load_gather(atab, [jnp.full((16,), 0, jnp.int32) + 2 * i])
            bv = plsc.load_gather(atab, [jnp.full((16,), 1, jnp.int32) + 2 * i])
            src = jnp.where(iota < 8, av * 8 + iota, 16000 + bv * 8 + iota - 8)
            obuf[pl.ds(g * 16, 16)] = plsc.load_gather(tab, [src])

        pltpu.sync_copy(obuf, out_hbm.at[pl.ds(tile * 128, 128)])

    return k(ops2000_flat, res_flat, actions_flat)


def _tc_head(ops, res, feat16, p):
    wa = _wlist(p, 'actor', [''])
    wc = _wlist(p, 'critic', [''])
    A = feat16.shape[0]

    def body(ops_ref, res_ref, f_ref, *rest):
        warefs = rest[0:6]
        wcrefs = rest[6:12]
        out_ref = rest[12]
        gs_o = jnp.mean(ops_ref[...], axis=0)
        gs_r = jnp.mean(res_ref[...], axis=0)
        gs = jnp.concatenate([gs_o, gs_r])[None, :]
        feat = jnp.concatenate(
            [f_ref[...], jnp.broadcast_to(gs, (A, 16))], -1)
        logits = _mlp_refs(feat, warefs, jnp.tanh)
        value = _mlp_refs(gs, wcrefs, jnp.tanh)
        out_ref[...] = jnp.concatenate(
            [logits[:, 0], value[:, 0]])

    return pl.pallas_call(
        body,
        in_specs=[pl.BlockSpec(ops.shape, lambda: (0, 0)),
                  pl.BlockSpec(res.shape, lambda: (0, 0)),
                  pl.BlockSpec(feat16.shape, lambda: (0, 0))]
        + [pl.BlockSpec(w.shape, lambda: (0, 0)) for w in wa + wc],
        out_specs=pl.BlockSpec((A + 1,), lambda: (0,)),
        out_shape=jax.ShapeDtypeStruct((A + 1,), jnp.float32),
    )(ops, res, feat16, *(wa + wc))


def _res_layer(resources, operations, req, p, pre):
    r = resources @ p[pre + 'Wr']
    o2000 = operations[:2000] @ p[pre + 'Wo']
    ac = p[pre + 'a_cross']
    uc = (r @ ac[:8]).reshape(-1)
    vc = (o2000 @ ac[8:]).reshape(-1)
    sa = jax.nn.leaky_relu(jnp.concatenate([r, r], -1) @ p[pre + 'a_self'], 0.2)
    out = _sc_res_edges(uc, vc, o2000.reshape(-1), req[0], req[1])
    SEGR = 16384 + 256 + 128
    o0, o1 = out[:SEGR], out[SEGR:]
    acc0 = o0[:16000].reshape(2000, 8)
    acc1 = o1[:16000].reshape(2000, 8)
    z0 = o0[16384:16640].sum()
    z1 = o1[16384:16640].sum()
    m0 = o0[16640]
    m1 = o1[16640]
    m = jnp.maximum(jnp.maximum(m0, m1), sa.max())
    e0 = jnp.exp(m0 - m)
    e1 = jnp.exp(m1 - m)
    esa = jnp.exp(sa - m)
    Z = z0 * e0 + z1 * e1 + esa.sum()
    summed = (acc0 * e0 + acc1 * e1) / Z
    ns = esa / Z
    return jax.nn.elu(ns * r + summed)


def _op_layer(operations, resources, prec, req, rep, p, pre):
    N, F = operations.shape
    parts = _sc_agg(resources.reshape(-1), req[0], req[1])
    agg2000 = (parts[:16000] + parts[16384:16384 + 16000]).reshape(2000, 8)
    agg = jnp.concatenate(
        [agg2000, jnp.zeros((N - 2000, 8), operations.dtype)], axis=0)
    NP, NP8 = 10240, 81920
    SEG = 2 * NP8 + 2 * NP
    pp = _sc_prec(operations.reshape(-1), prec[0], prec[1], rep, N, F)
    tot = pp[:SEG] + pp[SEG:]
    in_sum = tot[0:N * F].reshape(N, F)
    out_sum = tot[NP8:NP8 + N * F].reshape(N, F)
    in_cnt = tot[2 * NP8:2 * NP8 + N]
    out_cnt = tot[2 * NP8 + NP:2 * NP8 + NP + N]
    return _tc_op_mlps(in_sum, in_cnt, out_sum, out_cnt, operations, agg, p, pre)


def kernel(operations, resources, precedence_edges, requirement_edges, actions, params):
    p = params
    prec, req = precedence_edges, requirement_edges
    N = operations.shape[0]
    rep = _sc_rep_scatter(prec[0], prec[1], N)

    ops, res = operations, resources
    for l in range(2):
        res = _res_layer(res, ops, req, p, 'r%d_' % l)
        ops = _op_layer(ops, res, prec, req, rep, p, 'o%d_' % l)
    feat16 = _sc_action_gather(ops[:2000].reshape(-1), res.reshape(-1),
                               actions.reshape(-1)).reshape(256, 16)
    return _tc_head(ops, res, feat16, p)
